# Initial kernel scaffold; baseline (speedup 1.0000x reference)
import jax
import jax.numpy as jnp
from jax.experimental import pallas as pl


def kernel(word, dist, iniPos, word_table, dist_table):
    raise NotImplementedError("write your pallas kernel here")


# speedup vs baseline: 14.6226x; 14.6226x over previous
import functools

import jax
import jax.numpy as jnp
from jax import lax
from jax.experimental import pallas as pl
from jax.experimental.pallas import tpu as pltpu
from jax.experimental.pallas import tpu_sc as plsc

B, L = 4096, 50
VD = 1000
DW, DD = 128, 32
WIN = 2
NLOC = 2 * WIN + 1
DOUT = DW + DD + NLOC * DW
LP = 64
NC, NS, LANES = 2, 16, 16
NW = NC * NS
BPW = B // NW
GE = 8
NG = BPW // GE


def _lane_shuffle(v, idx):
    dnums = lax.GatherDimensionNumbers(
        offset_dims=(), collapsed_slice_dims=(0,), start_index_map=(0,))
    return lax.gather(v, idx[:, None], dnums, (1,),
                      mode=lax.GatherScatterMode.PROMISE_IN_BOUNDS)


def _sc_body(word_hbm, dist_hbm, pos_hbm, wtab_hbm, dtab_hbm, out_hbm,
             widx_v, didx_v, pos_v, w0_v, dtab_v, wrows_v, orow_v, sem):
    wid = lax.axis_index("s") * NC + lax.axis_index("c")
    base = wid * BPW
    pltpu.sync_copy(word_hbm.at[pl.ds(base, BPW)], widx_v)
    pltpu.sync_copy(dist_hbm.at[pl.ds(base, BPW)], didx_v)
    pltpu.sync_copy(pos_hbm.at[pl.ds(base, BPW)], pos_v.at[pl.ds(0, BPW)])
    pltpu.sync_copy(wtab_hbm.at[pl.ds(0, 1)], w0_v)
    pltpu.sync_copy(dtab_hbm, dtab_v)
    lanes = lax.iota(jnp.int32, LANES)
    zero = jnp.zeros((LANES,), jnp.float32)

    def grp(g, carry):
        gb = pl.multiple_of(g * GE, GE)
        copies = []
        for j in range(GE):
            cp = pltpu.make_async_copy(
                wtab_hbm.at[widx_v.at[gb + j, pl.ds(0, L)]],
                wrows_v.at[j], sem)
            cp.start()
            copies.append(cp)
        avec = pos_v[pl.ds(gb, LANES)]
        for j in range(GE):
            e = gb + j
            n0i = jnp.zeros((LANES,), jnp.int32)
            chunks = []
            for c in range(LP // LANES):
                wv = widx_v[e, pl.ds(c * LANES, LANES)]
                dv = didx_v[e, pl.ds(c * LANES, LANES)]
                z = wv == 0
                chunks.append(jnp.where(z, 0, dv))
                n0i = n0i + jnp.where(z, 1, 0)
            for s in (1, 2, 4, 8):
                n0i = n0i + _lane_shuffle(n0i, lanes ^ s)
            n0v = n0i.astype(jnp.float32)
            invv = 1.0 / jnp.maximum(jnp.float32(L) - n0v, 1.0)
            dacc = [zero for _ in range(DD // LANES)]
            for l in range(L):
                off = chunks[l // LANES][l % LANES] * DD
                for c in range(DD // LANES):
                    dacc[c] = dacc[c] + dtab_v[pl.ds(off + c * LANES, LANES)]
            for c in range(DD // LANES):
                sd = (dacc[c] - n0v * dtab_v[pl.ds(c * LANES, LANES)]) * invv
                orow_v[j, pl.ds(DW + c * LANES, LANES)] = sd
            copies[j].wait()
            def wsum(l, accs):
                return tuple(
                    accs[c] + wrows_v[j, l, pl.ds(c * LANES, LANES)]
                    for c in range(DW // LANES))
            accs = lax.fori_loop(0, L, wsum,
                                 tuple(zero for _ in range(DW // LANES)))
            for c in range(DW // LANES):
                sw = (accs[c] - n0v * w0_v[0, pl.ds(c * LANES, LANES)]) * invv
                orow_v[j, pl.ds(c * LANES, LANES)] = sw
            a = avec[j]
            for i in range(NLOC):
                p = a + (i - WIN)
                valid = jnp.where((p >= 0) & (p < L), 1.0, 0.0)
                vsplat = jnp.full((LANES,), valid, jnp.float32)
                pc = jnp.clip(p, 0, L - 1)
                for c in range(DW // LANES):
                    v = wrows_v[j, pc, pl.ds(c * LANES, LANES)]
                    orow_v[j, pl.ds(DW + DD + i * DW + c * LANES, LANES)] = (
                        v * vsplat)
        pltpu.sync_copy(orow_v, out_hbm.at[pl.ds(base + gb, GE)])
        return carry

    lax.fori_loop(0, NG, grp, 0)


_mesh = plsc.VectorSubcoreMesh(core_axis_name="c", subcore_axis_name="s")


@functools.partial(
    pl.kernel,
    mesh=_mesh,
    out_type=jax.ShapeDtypeStruct((B, DOUT), jnp.float32),
    scratch_types=[
        pltpu.VMEM((BPW, LP), jnp.int32),
        pltpu.VMEM((BPW, LP), jnp.int32),
        pltpu.VMEM((BPW + LANES,), jnp.int32),
        pltpu.VMEM((1, DW), jnp.float32),
        pltpu.VMEM((VD * DD,), jnp.float32),
        pltpu.VMEM((GE, L, DW), jnp.float32),
        pltpu.VMEM((GE, DOUT), jnp.float32),
        pltpu.SemaphoreType.DMA,
    ],
)
def _sc_kernel(*refs):
    _sc_body(*refs)


def kernel(word, dist, iniPos, word_table, dist_table):
    word = word.astype(jnp.int32)
    dist = dist.astype(jnp.int32)
    pad_w = jnp.ones((B, LP - L), jnp.int32)
    pad_d = jnp.zeros((B, LP - L), jnp.int32)
    wp = jnp.concatenate([word, pad_w], axis=1)
    dp = jnp.concatenate([dist, pad_d], axis=1)
    return _sc_kernel(wp, dp, iniPos.astype(jnp.int32), word_table,
                      dist_table.reshape(-1))


# speedup vs baseline: 15.9513x; 1.0909x over previous
import functools

import jax
import jax.numpy as jnp
from jax import lax
from jax.experimental import pallas as pl
from jax.experimental.pallas import tpu as pltpu
from jax.experimental.pallas import tpu_sc as plsc

B, L = 4096, 50
VD = 1000
DW, DD = 128, 32
WIN = 2
NLOC = 2 * WIN + 1
DOUT = DW + DD + NLOC * DW
LP = 64
NC, NS, LANES = 2, 16, 16
NW = NC * NS
BPW = B // NW
GE = 4
NG = BPW // GE


def _lane_shuffle(v, idx):
    dnums = lax.GatherDimensionNumbers(
        offset_dims=(), collapsed_slice_dims=(0,), start_index_map=(0,))
    return lax.gather(v, idx[:, None], dnums, (1,),
                      mode=lax.GatherScatterMode.PROMISE_IN_BOUNDS)


def _sc_body(word_hbm, dist_hbm, pos_hbm, wtab_hbm, dtab_hbm, out_hbm,
             widx_v, didx_v, pos_v, w0_v, dtab_v, wrows, orow,
             sem_a, sem_b, osem_a, osem_b):
    sems = (sem_a, sem_b)
    osems = (osem_a, osem_b)
    wid = lax.axis_index("s") * NC + lax.axis_index("c")
    base = wid * BPW
    pltpu.sync_copy(word_hbm.at[pl.ds(base, BPW)], widx_v)
    pltpu.sync_copy(dist_hbm.at[pl.ds(base, BPW)], didx_v)
    pltpu.sync_copy(pos_hbm.at[pl.ds(base, BPW)], pos_v.at[pl.ds(0, BPW)])
    pltpu.sync_copy(wtab_hbm.at[pl.ds(0, 1)], w0_v)
    pltpu.sync_copy(dtab_hbm, dtab_v)
    lanes = lax.iota(jnp.int32, LANES)
    zero = jnp.zeros((LANES,), jnp.float32)

    def gather_group(g, buf):
        for j in range(GE):
            pltpu.make_async_copy(
                wtab_hbm.at[widx_v.at[g * GE + j, pl.ds(0, L)]],
                wrows.at[buf, j], sems[buf]).start()

    def drain_group(buf):
        for j in range(GE):
            pltpu.make_async_copy(
                wtab_hbm.at[widx_v.at[0, pl.ds(0, L)]],
                wrows.at[buf, j], sems[buf]).wait()

    def compute_group(g, buf, k):
        gb = pl.multiple_of(g * GE, GE)
        avec = pos_v[pl.ds(gb, LANES)]
        @pl.when(k > 0)
        def _():
            pltpu.make_async_copy(
                orow.at[buf], out_hbm.at[pl.ds(base + gb, GE)],
                osems[buf]).wait()
        drain_group(buf)
        for j in range(GE):
            e = gb + j
            n0i = jnp.zeros((LANES,), jnp.int32)
            chunks = []
            for c in range(LP // LANES):
                wv = widx_v[e, pl.ds(c * LANES, LANES)]
                dv = didx_v[e, pl.ds(c * LANES, LANES)]
                z = wv == 0
                chunks.append(jnp.where(z, 0, dv))
                n0i = n0i + jnp.where(z, 1, 0)
            for s in (1, 2, 4, 8):
                n0i = n0i + _lane_shuffle(n0i, lanes ^ s)
            n0v = n0i.astype(jnp.float32)
            invv = 1.0 / jnp.maximum(jnp.float32(L) - n0v, 1.0)
            dacc = [zero for _ in range(DD // LANES)]
            for l in range(L):
                off = chunks[l // LANES][l % LANES] * DD
                for c in range(DD // LANES):
                    dacc[c] = dacc[c] + dtab_v[pl.ds(off + c * LANES, LANES)]
            for c in range(DD // LANES):
                sd = (dacc[c] - n0v * dtab_v[pl.ds(c * LANES, LANES)]) * invv
                orow[buf, j, pl.ds(DW + c * LANES, LANES)] = sd
            def wsum(l, accs):
                return tuple(
                    accs[c] + wrows[buf, j, l, pl.ds(c * LANES, LANES)]
                    for c in range(DW // LANES))
            accs = lax.fori_loop(0, L, wsum,
                                 tuple(zero for _ in range(DW // LANES)))
            for c in range(DW // LANES):
                sw = (accs[c] - n0v * w0_v[0, pl.ds(c * LANES, LANES)]) * invv
                orow[buf, j, pl.ds(c * LANES, LANES)] = sw
            a = avec[j]
            for i in range(NLOC):
                p = a + (i - WIN)
                valid = jnp.where((p >= 0) & (p < L), 1.0, 0.0)
                vsplat = jnp.full((LANES,), valid, jnp.float32)
                pc = jnp.clip(p, 0, L - 1)
                for c in range(DW // LANES):
                    v = wrows[buf, j, pc, pl.ds(c * LANES, LANES)]
                    orow[buf, j,
                         pl.ds(DW + DD + i * DW + c * LANES, LANES)] = (
                        v * vsplat)
        pltpu.make_async_copy(
            orow.at[buf], out_hbm.at[pl.ds(base + gb, GE)],
            osems[buf]).start()

    gather_group(0, 0)

    def iter_k(k, carry):
        g0 = 2 * k
        gather_group(g0 + 1, 1)
        compute_group(g0, 0, k)
        gnext = jnp.minimum(g0 + 2, NG - 1)
        gather_group(gnext, 0)
        compute_group(g0 + 1, 1, k)
        return carry

    lax.fori_loop(0, NG // 2, iter_k, 0)
    drain_group(0)
    for buf in range(2):
        pltpu.make_async_copy(
            orow.at[buf], out_hbm.at[pl.ds(base, GE)], osems[buf]).wait()


_mesh = plsc.VectorSubcoreMesh(core_axis_name="c", subcore_axis_name="s")


@functools.partial(
    pl.kernel,
    mesh=_mesh,
    out_type=jax.ShapeDtypeStruct((B, DOUT), jnp.float32),
    scratch_types=[
        pltpu.VMEM((BPW, LP), jnp.int32),
        pltpu.VMEM((BPW, LP), jnp.int32),
        pltpu.VMEM((BPW + LANES,), jnp.int32),
        pltpu.VMEM((1, DW), jnp.float32),
        pltpu.VMEM((VD * DD,), jnp.float32),
        pltpu.VMEM((2, GE, L, DW), jnp.float32),
        pltpu.VMEM((2, GE, DOUT), jnp.float32),
        pltpu.SemaphoreType.DMA,
        pltpu.SemaphoreType.DMA,
        pltpu.SemaphoreType.DMA,
        pltpu.SemaphoreType.DMA,
    ],
)
def _sc_kernel(*refs):
    _sc_body(*refs)


def kernel(word, dist, iniPos, word_table, dist_table):
    word = word.astype(jnp.int32)
    dist = dist.astype(jnp.int32)
    pad_w = jnp.ones((B, LP - L), jnp.int32)
    pad_d = jnp.zeros((B, LP - L), jnp.int32)
    wp = jnp.concatenate([word, pad_w], axis=1)
    dp = jnp.concatenate([dist, pad_d], axis=1)
    return _sc_kernel(wp, dp, iniPos.astype(jnp.int32), word_table,
                      dist_table.reshape(-1))


# speedup vs baseline: 16.0387x; 1.0055x over previous
import functools

import jax
import jax.numpy as jnp
from jax import lax
from jax.experimental import pallas as pl
from jax.experimental.pallas import tpu as pltpu
from jax.experimental.pallas import tpu_sc as plsc

B, L = 4096, 50
VD = 1000
DW, DD = 128, 32
WIN = 2
NLOC = 2 * WIN + 1
DOUT = DW + DD + NLOC * DW
NC, NS, LANES = 2, 16, 16
NW = NC * NS
BPW = B // NW
GE = 4
GL = GE * L
S1 = 96
NG = BPW // GE
COFFS = (0, 16, 32, 34)


def _lane_shuffle(v, idx):
    dnums = lax.GatherDimensionNumbers(
        offset_dims=(), collapsed_slice_dims=(0,), start_index_map=(0,))
    return lax.gather(v, idx[:, None], dnums, (1,),
                      mode=lax.GatherScatterMode.PROMISE_IN_BOUNDS)


def _chunk_of(l):
    if l < 32:
        return l // 16, l % 16
    if l < 48:
        return 2, l - 32
    return 3, l - 34


def _sc_body(word_hbm, dist_hbm, pos_hbm, wtab_hbm, dtab_hbm, out_hbm,
             widx_v, didx_v, pos_v, w0_v, dtab_v, wrows, orow,
             sem_a, sem_b, osem_a, osem_b):
    sems = (sem_a, sem_b)
    osems = (osem_a, osem_b)
    wid = lax.axis_index("s") * NC + lax.axis_index("c")
    base = wid * BPW
    pltpu.sync_copy(word_hbm.at[pl.ds(base * L, BPW * L)], widx_v)
    pltpu.sync_copy(dist_hbm.at[pl.ds(base * L, BPW * L)], didx_v)
    pltpu.sync_copy(pos_hbm.at[pl.ds(base, BPW)], pos_v.at[pl.ds(0, BPW)])
    pltpu.sync_copy(wtab_hbm.at[pl.ds(0, 1)], w0_v)
    pltpu.sync_copy(dtab_hbm, dtab_v)
    lanes = lax.iota(jnp.int32, LANES)
    zero = jnp.zeros((LANES,), jnp.float32)

    def gather_group(g, buf):
        gb = pl.multiple_of(g * GL, 8)
        pltpu.make_async_copy(
            wtab_hbm.at[widx_v.at[pl.ds(gb, S1)]],
            wrows.at[buf, pl.ds(0, S1)], sems[buf]).start()
        pltpu.make_async_copy(
            wtab_hbm.at[widx_v.at[pl.ds(gb + S1, GL - S1)]],
            wrows.at[buf, pl.ds(S1, GL - S1)], sems[buf]).start()

    def drain_group(buf):
        pltpu.make_async_copy(
            wtab_hbm.at[widx_v.at[pl.ds(0, S1)]],
            wrows.at[buf, pl.ds(0, S1)], sems[buf]).wait()
        pltpu.make_async_copy(
            wtab_hbm.at[widx_v.at[pl.ds(0, GL - S1)]],
            wrows.at[buf, pl.ds(S1, GL - S1)], sems[buf]).wait()

    def compute_group(g, buf, k):
        gb = pl.multiple_of(g * GE, GE)
        avec = pos_v[pl.ds(gb, LANES)]
        @pl.when(k > 0)
        def _():
            pltpu.make_async_copy(
                orow.at[buf], out_hbm.at[pl.ds(base + gb, GE)],
                osems[buf]).wait()
        drain_group(buf)
        for j in range(GE):
            eoff = g * GL + j * L
            n0i = jnp.zeros((LANES,), jnp.int32)
            chunks = []
            for ci, co in enumerate(COFFS):
                wv = widx_v[pl.ds(eoff + co, LANES)]
                dv = didx_v[pl.ds(eoff + co, LANES)]
                z = wv == 0
                chunks.append(jnp.where(z, 0, dv))
                if ci < 3:
                    n0i = n0i + jnp.where(z, 1, 0)
                else:
                    n0i = n0i + jnp.where(z & (lanes >= LANES - 2), 1, 0)
            for s in (1, 2, 4, 8):
                n0i = n0i + _lane_shuffle(n0i, lanes ^ s)
            n0v = n0i.astype(jnp.float32)
            invv = 1.0 / jnp.maximum(jnp.float32(L) - n0v, 1.0)
            dacc = [zero for _ in range(DD // LANES)]
            for l in range(L):
                ci, lane = _chunk_of(l)
                off = chunks[ci][lane] * DD
                for c in range(DD // LANES):
                    dacc[c] = dacc[c] + dtab_v[pl.ds(off + c * LANES, LANES)]
            for c in range(DD // LANES):
                sd = (dacc[c] - n0v * dtab_v[pl.ds(c * LANES, LANES)]) * invv
                orow[buf, j, pl.ds(DW + c * LANES, LANES)] = sd
            def wsum(l, accs):
                return tuple(
                    accs[c] + wrows[buf, j * L + l, pl.ds(c * LANES, LANES)]
                    for c in range(DW // LANES))
            accs = lax.fori_loop(0, L, wsum,
                                 tuple(zero for _ in range(DW // LANES)))
            for c in range(DW // LANES):
                sw = (accs[c] - n0v * w0_v[0, pl.ds(c * LANES, LANES)]) * invv
                orow[buf, j, pl.ds(c * LANES, LANES)] = sw
            a = avec[j]
            for i in range(NLOC):
                p = a + (i - WIN)
                valid = jnp.where((p >= 0) & (p < L), 1.0, 0.0)
                vsplat = jnp.full((LANES,), valid, jnp.float32)
                pc = jnp.clip(p, 0, L - 1)
                for c in range(DW // LANES):
                    v = wrows[buf, j * L + pc, pl.ds(c * LANES, LANES)]
                    orow[buf, j,
                         pl.ds(DW + DD + i * DW + c * LANES, LANES)] = (
                        v * vsplat)
        pltpu.make_async_copy(
            orow.at[buf], out_hbm.at[pl.ds(base + gb, GE)],
            osems[buf]).start()

    gather_group(0, 0)

    def iter_k(k, carry):
        g0 = 2 * k
        gather_group(g0 + 1, 1)
        compute_group(g0, 0, k)
        gnext = jnp.minimum(g0 + 2, NG - 1)
        gather_group(gnext, 0)
        compute_group(g0 + 1, 1, k)
        return carry

    lax.fori_loop(0, NG // 2, iter_k, 0)
    drain_group(0)
    for buf in range(2):
        pltpu.make_async_copy(
            orow.at[buf], out_hbm.at[pl.ds(base, GE)], osems[buf]).wait()


_mesh = plsc.VectorSubcoreMesh(core_axis_name="c", subcore_axis_name="s")


@functools.partial(
    pl.kernel,
    mesh=_mesh,
    out_type=jax.ShapeDtypeStruct((B, DOUT), jnp.float32),
    scratch_types=[
        pltpu.VMEM((BPW * L,), jnp.int32),
        pltpu.VMEM((BPW * L,), jnp.int32),
        pltpu.VMEM((BPW + LANES,), jnp.int32),
        pltpu.VMEM((1, DW), jnp.float32),
        pltpu.VMEM((VD * DD,), jnp.float32),
        pltpu.VMEM((2, GL, DW), jnp.float32),
        pltpu.VMEM((2, GE, DOUT), jnp.float32),
        pltpu.SemaphoreType.DMA,
        pltpu.SemaphoreType.DMA,
        pltpu.SemaphoreType.DMA,
        pltpu.SemaphoreType.DMA,
    ],
)
def _sc_kernel(*refs):
    _sc_body(*refs)


def kernel(word, dist, iniPos, word_table, dist_table):
    return _sc_kernel(word.astype(jnp.int32).reshape(-1),
                      dist.astype(jnp.int32).reshape(-1),
                      iniPos.astype(jnp.int32), word_table,
                      dist_table.reshape(-1))
